# baseline (speedup 1.0000x reference)
import functools

import jax
import jax.numpy as jnp
from jax import lax
from jax.experimental import pallas as pl
from jax.experimental.pallas import tpu as pltpu
from jax.experimental.pallas import tpu_sc as plsc

N = 10000
NP = 10240
E = 320000
D_IN = 128
PDIM = 16
HID = 128
D = 128
DH = 64
NC = 2
NS = 16
L = 16
NW = NC * NS
EPW = E // NW
CH = 125
NCH = EPW // CH
RING = 4
RPT = NP // NS
DW = 16
GPW = EPW // L

_mesh = plsc.VectorSubcoreMesh(core_axis_name="c", subcore_axis_name="s")


@functools.partial(
    pl.kernel,
    out_type=jax.ShapeDtypeStruct((NC, NP, DW), jnp.float32),
    mesh=_mesh,
    compiler_params=pltpu.CompilerParams(use_tc_tiling_on_sc=False, needs_layout_passes=False),
    scratch_types=[
        pltpu.VMEM((NCH, CH), jnp.int32),
        pltpu.VMEM((CH, DW), jnp.float32),
        pltpu.VMEM_SHARED((NP, DW), jnp.float32),
    ] + [pltpu.SemaphoreType.DMA] * 2,
)
def _deg_kernel(dst_hbm, ones_hbm, zeros_hbm, degp_hbm, idx_v, ones_v, acc_sh,
                *sems):
    cid = lax.axis_index("c")
    sid = lax.axis_index("s")
    wid = sid * NC + cid
    pltpu.sync_copy(dst_hbm.at[wid], idx_v)
    pltpu.sync_copy(ones_hbm, ones_v)
    pltpu.sync_copy(zeros_hbm.at[pl.ds(sid * RPT, RPT)],
                    acc_sh.at[pl.ds(sid * RPT, RPT)])
    plsc.subcore_barrier()

    def scat(j, k):
        pltpu.make_async_copy(ones_v, acc_sh.at[idx_v.at[j]], sems[k]).start(add=True)

    def scat_wait(j, k):
        pltpu.make_async_copy(ones_v, acc_sh.at[idx_v.at[j]], sems[k]).wait()

    scat(0, 0)
    scat(1, 1)

    def body(i, carry):
        j0 = i * 2
        scat_wait(j0 - 2, 0)
        scat(j0, 0)
        scat_wait(j0 - 1, 1)
        scat(j0 + 1, 1)
        return carry

    lax.fori_loop(1, NCH // 2, body, 0)
    scat_wait(NCH - 2, 0)
    scat_wait(NCH - 1, 1)
    plsc.subcore_barrier()
    pltpu.sync_copy(acc_sh.at[pl.ds(sid * RPT, RPT)],
                    degp_hbm.at[cid, pl.ds(sid * RPT, RPT)])


@functools.partial(
    pl.kernel,
    out_type=[
        jax.ShapeDtypeStruct((NC, NP, DH), jnp.float32),
        jax.ShapeDtypeStruct((NC, NP, DH), jnp.float32),
    ],
    mesh=_mesh,
    compiler_params=pltpu.CompilerParams(use_tc_tiling_on_sc=False, needs_layout_passes=False),
    scratch_types=[
        pltpu.VMEM((NCH, CH), jnp.int32),
        pltpu.VMEM((NCH, CH), jnp.int32),
    ] + [pltpu.VMEM((CH, DH), jnp.float32)] * RING
      + [pltpu.VMEM_SHARED((NP, DH), jnp.float32)]
      + [pltpu.SemaphoreType.DMA] * (2 * RING),
)
def _scatter_kernel(ul_hbm, ur_hbm, src_hbm, dst_hbm, zeros_hbm,
                    pl_hbm, pr_hbm,
                    src_v, dst_v, *rest):
    bufs = rest[:RING]
    acc_sh = rest[RING]
    gsem = rest[RING + 1:RING + 1 + RING]
    ssem = rest[RING + 1 + RING:]
    cid = lax.axis_index("c")
    sid = lax.axis_index("s")
    wid = sid * NC + cid
    pltpu.sync_copy(src_hbm.at[wid], src_v)
    pltpu.sync_copy(dst_hbm.at[wid], dst_v)

    HL = RING // 2

    for u_hbm, out_hbm in ((ul_hbm, pl_hbm), (ur_hbm, pr_hbm)):
        pltpu.sync_copy(zeros_hbm.at[pl.ds(sid * RPT, RPT)],
                        acc_sh.at[pl.ds(sid * RPT, RPT)])
        plsc.subcore_barrier()

        def gather(j, k):
            pltpu.make_async_copy(u_hbm.at[src_v.at[j]], bufs[k], gsem[k]).start()

        def gather_wait(j, k):
            pltpu.make_async_copy(u_hbm.at[src_v.at[j]], bufs[k], gsem[k]).wait()

        def scat(j, k):
            pltpu.make_async_copy(
                bufs[k], acc_sh.at[dst_v.at[j]], ssem[k]).start(add=True)

        def scat_wait(j, k):
            pltpu.make_async_copy(
                bufs[k], acc_sh.at[dst_v.at[j]], ssem[k]).wait()

        def slot(j, k, wait_s, fire_g):
            gather_wait(j, k)
            scat(j, k)
            if wait_s:
                scat_wait(j - HL, (k - HL) % RING)
            if fire_g:
                gather(j + HL, (k + HL) % RING)

        for j in range(HL):
            gather(j, j)
        for j in range(RING):
            slot(j, j, wait_s=(j >= HL), fire_g=True)

        def body(i, carry):
            j0 = i * RING
            for k in range(RING):
                slot(j0 + k, k, wait_s=True, fire_g=True)
            return carry

        lax.fori_loop(1, NCH // RING - 1, body, 0)
        e = NCH - RING
        for j in range(e, NCH):
            slot(j, j % RING, wait_s=True, fire_g=(j + HL < NCH))
        for j in range(NCH - HL, NCH):
            scat_wait(j, j % RING)
        plsc.subcore_barrier()
        pltpu.sync_copy(acc_sh.at[pl.ds(sid * RPT, RPT)],
                        out_hbm.at[cid, pl.ds(sid * RPT, RPT)])


@functools.partial(
    pl.kernel,
    out_type=jax.ShapeDtypeStruct((NW, GPW, L), jnp.float32),
    mesh=_mesh,
    compiler_params=pltpu.CompilerParams(use_tc_tiling_on_sc=False, needs_layout_passes=False),
    scratch_types=[
        pltpu.VMEM((N, 2), jnp.float32),
        pltpu.VMEM((GPW, L), jnp.int32),
        pltpu.VMEM((GPW, L), jnp.int32),
        pltpu.VMEM((GPW, L), jnp.float32),
        pltpu.VMEM((L,), jnp.float32),
    ],
)
def _decode_kernel(ac_hbm, ls_hbm, ld_hbm, bl_hbm, out_hbm,
                   ac_v, ls_v, ld_v, o_v, bl_v):
    cid = lax.axis_index("c")
    sid = lax.axis_index("s")
    wid = sid * NC + cid
    pltpu.sync_copy(ac_hbm, ac_v)
    pltpu.sync_copy(ls_hbm.at[wid], ls_v)
    pltpu.sync_copy(ld_hbm.at[wid], ld_v)
    pltpu.sync_copy(bl_hbm, bl_v)
    bv = bl_v[...]
    col_a = jnp.zeros((L,), jnp.int32)
    col_c = jnp.ones((L,), jnp.int32)

    def body(i, carry):
        j0 = i * 5
        for k in range(5):
            j = j0 + k
            si = ls_v[j]
            di = ld_v[j]
            av = plsc.load_gather(ac_v, [si, col_a])
            cv = plsc.load_gather(ac_v, [di, col_c])
            o_v[j] = av + cv + bv
        return carry

    lax.fori_loop(0, GPW // 5, body, 0)
    pltpu.sync_copy(o_v, out_hbm.at[wid])


def _dinv_of(degp_ref):
    d = degp_ref[...]
    deg = jnp.sum(d[0] + d[1], axis=-1) * (1.0 / DW) + 1.0
    return lax.rsqrt(deg)


def _tca_body(x_ref, pe_ref, w1_ref, degp_ref, ul_ref, ur_ref):
    dinv = _dinv_of(degp_ref)
    xw = jnp.dot(x_ref[...], w1_ref[:D_IN, :], preferred_element_type=jnp.float32)
    xw = xw + jnp.dot(pe_ref[...], w1_ref[D_IN:, :], preferred_element_type=jnp.float32)
    u = xw * dinv[:, None]
    ul_ref[...] = u[:, :DH]
    ur_ref[...] = u[:, DH:]


def _tcb_body(ul_ref, ur_ref, pl_ref, pr_ref, degp_ref, b1_ref, w2_ref,
              ol_ref, or_ref):
    dinv = _dinv_of(degp_ref)
    aggl = ul_ref[...] + pl_ref[0] + pl_ref[1]
    aggr = ur_ref[...] + pr_ref[0] + pr_ref[1]
    agg = jnp.concatenate([aggl, aggr], axis=-1)
    h = jnp.maximum(agg * dinv[:, None] + b1_ref[...][None, :], 0.0)
    u2 = jnp.dot(h, w2_ref[...], preferred_element_type=jnp.float32) * dinv[:, None]
    ol_ref[...] = u2[:, :DH]
    or_ref[...] = u2[:, DH:]


def _tcc_body(ul_ref, ur_ref, pl_ref, pr_ref, degp_ref, b2_ref, wac_ref,
              z_ref, ac_ref):
    dinv = _dinv_of(degp_ref)
    aggl = ul_ref[...] + pl_ref[0] + pl_ref[1]
    aggr = ur_ref[...] + pr_ref[0] + pr_ref[1]
    agg = jnp.concatenate([aggl, aggr], axis=-1)
    z = agg * dinv[:, None] + b2_ref[...][None, :]
    z_ref[...] = z
    ac_ref[...] = jnp.dot(z, wac_ref[...], preferred_element_type=jnp.float32)


BLK = 5000

_tc_a = pl.pallas_call(
    _tca_body,
    grid=(N // BLK,),
    in_specs=[
        pl.BlockSpec((BLK, D_IN), lambda i: (i, 0)),
        pl.BlockSpec((BLK, PDIM), lambda i: (i, 0)),
        pl.BlockSpec((D_IN + PDIM, HID), lambda i: (0, 0)),
        pl.BlockSpec((NC, BLK, DW), lambda i: (0, i, 0)),
    ],
    out_specs=[
        pl.BlockSpec((BLK, DH), lambda i: (i, 0)),
        pl.BlockSpec((BLK, DH), lambda i: (i, 0)),
    ],
    out_shape=[
        jax.ShapeDtypeStruct((N, DH), jnp.float32),
        jax.ShapeDtypeStruct((N, DH), jnp.float32),
    ],
)

_tc_b = pl.pallas_call(
    _tcb_body,
    grid=(N // BLK,),
    in_specs=[
        pl.BlockSpec((BLK, DH), lambda i: (i, 0)),
        pl.BlockSpec((BLK, DH), lambda i: (i, 0)),
        pl.BlockSpec((NC, BLK, DH), lambda i: (0, i, 0)),
        pl.BlockSpec((NC, BLK, DH), lambda i: (0, i, 0)),
        pl.BlockSpec((NC, BLK, DW), lambda i: (0, i, 0)),
        pl.BlockSpec((HID,), lambda i: (0,)),
        pl.BlockSpec((HID, D), lambda i: (0, 0)),
    ],
    out_specs=[
        pl.BlockSpec((BLK, DH), lambda i: (i, 0)),
        pl.BlockSpec((BLK, DH), lambda i: (i, 0)),
    ],
    out_shape=[
        jax.ShapeDtypeStruct((N, DH), jnp.float32),
        jax.ShapeDtypeStruct((N, DH), jnp.float32),
    ],
)

_tc_c = pl.pallas_call(
    _tcc_body,
    grid=(N // BLK,),
    in_specs=[
        pl.BlockSpec((BLK, DH), lambda i: (i, 0)),
        pl.BlockSpec((BLK, DH), lambda i: (i, 0)),
        pl.BlockSpec((NC, BLK, DH), lambda i: (0, i, 0)),
        pl.BlockSpec((NC, BLK, DH), lambda i: (0, i, 0)),
        pl.BlockSpec((NC, BLK, DW), lambda i: (0, i, 0)),
        pl.BlockSpec((D,), lambda i: (0,)),
        pl.BlockSpec((D, 2), lambda i: (0, 0)),
    ],
    out_specs=[
        pl.BlockSpec((BLK, D), lambda i: (i, 0)),
        pl.BlockSpec((BLK, 2), lambda i: (i, 0)),
    ],
    out_shape=[
        jax.ShapeDtypeStruct((N, D), jnp.float32),
        jax.ShapeDtypeStruct((N, 2), jnp.float32),
    ],
)


def kernel(x, edge_index, edge_label_index, pos_emb, W1, b1, W2, b2, Wl, bl):
    src_r = edge_index[0].reshape(NW, NCH, CH)
    dst_r = edge_index[1].reshape(NW, NCH, CH)
    ls_r = edge_label_index[0].reshape(NW, GPW, L)
    ld_r = edge_label_index[1].reshape(NW, GPW, L)
    zeros_half = jnp.zeros((NP, DH), jnp.float32)
    zeros_deg = jnp.zeros((NP, DW), jnp.float32)
    ones_deg = jnp.ones((CH, DW), jnp.float32)
    bl16 = jnp.full((L,), bl[0], jnp.float32)
    wac = jnp.stack([Wl[:HID, 0], Wl[HID:, 0]], axis=1)

    degp = _deg_kernel(dst_r, ones_deg, zeros_deg)
    u1l, u1r = _tc_a(x, pos_emb, W1, degp)
    p1l, p1r = _scatter_kernel(u1l, u1r, src_r, dst_r, zeros_half)
    u2l, u2r = _tc_b(u1l, u1r, p1l, p1r, degp, b1, W2)
    p2l, p2r = _scatter_kernel(u2l, u2r, src_r, dst_r, zeros_half)
    z, ac = _tc_c(u2l, u2r, p2l, p2r, degp, b2, wac)
    ep = _decode_kernel(ac, ls_r, ld_r, bl16).reshape(E)
    return (z, ep)


# speedup vs baseline: 1.0067x; 1.0067x over previous
import functools

import jax
import jax.numpy as jnp
from jax import lax
from jax.experimental import pallas as pl
from jax.experimental.pallas import tpu as pltpu
from jax.experimental.pallas import tpu_sc as plsc

N = 10000
NP = 10240
E = 320000
D_IN = 128
PDIM = 16
HID = 128
D = 128
DH = 64
NC = 2
NS = 16
L = 16
NW = NC * NS
EPW = E // NW
CH = 125
NCH = EPW // CH
RING = 4
RPT = NP // NS
DW = 16
GPW = EPW // L

_mesh = plsc.VectorSubcoreMesh(core_axis_name="c", subcore_axis_name="s")


@functools.partial(
    pl.kernel,
    out_type=jax.ShapeDtypeStruct((NC, NP, DW), jnp.float32),
    mesh=_mesh,
    compiler_params=pltpu.CompilerParams(use_tc_tiling_on_sc=False, needs_layout_passes=False),
    scratch_types=[
        pltpu.VMEM((NCH, CH), jnp.int32),
        pltpu.VMEM((CH, DW), jnp.float32),
        pltpu.VMEM_SHARED((NP, DW), jnp.float32),
    ] + [pltpu.SemaphoreType.DMA] * 2,
)
def _deg_kernel(dst_hbm, ones_hbm, zeros_hbm, degp_hbm, idx_v, ones_v, acc_sh,
                *sems):
    cid = lax.axis_index("c")
    sid = lax.axis_index("s")
    wid = sid * NC + cid
    pltpu.sync_copy(dst_hbm.at[wid], idx_v)
    pltpu.sync_copy(ones_hbm, ones_v)
    pltpu.sync_copy(zeros_hbm.at[pl.ds(sid * RPT, RPT)],
                    acc_sh.at[pl.ds(sid * RPT, RPT)])
    plsc.subcore_barrier()

    def scat(j, k):
        pltpu.make_async_copy(ones_v, acc_sh.at[idx_v.at[j]], sems[k]).start(add=True)

    def scat_wait(j, k):
        pltpu.make_async_copy(ones_v, acc_sh.at[idx_v.at[j]], sems[k]).wait()

    scat(0, 0)
    scat(1, 1)

    def body(i, carry):
        j0 = i * 2
        scat_wait(j0 - 2, 0)
        scat(j0, 0)
        scat_wait(j0 - 1, 1)
        scat(j0 + 1, 1)
        return carry

    lax.fori_loop(1, NCH // 2, body, 0)
    scat_wait(NCH - 2, 0)
    scat_wait(NCH - 1, 1)
    plsc.subcore_barrier()
    pltpu.sync_copy(acc_sh.at[pl.ds(sid * RPT, RPT)],
                    degp_hbm.at[cid, pl.ds(sid * RPT, RPT)])


@functools.partial(
    pl.kernel,
    out_type=[
        jax.ShapeDtypeStruct((NC, NP, DH), jnp.float32),
        jax.ShapeDtypeStruct((NC, NP, DH), jnp.float32),
    ],
    mesh=_mesh,
    compiler_params=pltpu.CompilerParams(use_tc_tiling_on_sc=False, needs_layout_passes=False),
    scratch_types=[
        pltpu.VMEM((NCH, CH), jnp.int32),
        pltpu.VMEM((NCH, CH), jnp.int32),
    ] + [pltpu.VMEM((CH, DH), jnp.float32)] * RING
      + [pltpu.VMEM_SHARED((NP, DH), jnp.float32)]
      + [pltpu.SemaphoreType.DMA] * (2 * RING),
)
def _scatter_kernel(ul_hbm, ur_hbm, src_hbm, dst_hbm, zeros_hbm,
                    pl_hbm, pr_hbm,
                    src_v, dst_v, *rest):
    bufs = rest[:RING]
    acc_sh = rest[RING]
    gsem = rest[RING + 1:RING + 1 + RING]
    ssem = rest[RING + 1 + RING:]
    cid = lax.axis_index("c")
    sid = lax.axis_index("s")
    wid = sid * NC + cid
    pltpu.sync_copy(src_hbm.at[wid], src_v)
    pltpu.sync_copy(dst_hbm.at[wid], dst_v)

    HL = RING // 2

    for u_hbm, out_hbm in ((ul_hbm, pl_hbm), (ur_hbm, pr_hbm)):
        pltpu.sync_copy(zeros_hbm.at[pl.ds(sid * RPT, RPT)],
                        acc_sh.at[pl.ds(sid * RPT, RPT)])
        plsc.subcore_barrier()

        def gather(j, k):
            pltpu.make_async_copy(u_hbm.at[src_v.at[j]], bufs[k], gsem[k]).start()

        def gather_wait(j, k):
            pltpu.make_async_copy(u_hbm.at[src_v.at[j]], bufs[k], gsem[k]).wait()

        def scat(j, k):
            pltpu.make_async_copy(
                bufs[k], acc_sh.at[dst_v.at[j]], ssem[k]).start(add=True)

        def scat_wait(j, k):
            pltpu.make_async_copy(
                bufs[k], acc_sh.at[dst_v.at[j]], ssem[k]).wait()

        def slot(j, k, wait_s, fire_g):
            gather_wait(j, k)
            scat(j, k)
            if wait_s:
                scat_wait(j - HL, (k - HL) % RING)
            if fire_g:
                gather(j + HL, (k + HL) % RING)

        for j in range(HL):
            gather(j, j)
        for j in range(RING):
            slot(j, j, wait_s=(j >= HL), fire_g=True)

        def body(i, carry):
            j0 = i * RING
            for k in range(RING):
                slot(j0 + k, k, wait_s=True, fire_g=True)
            return carry

        lax.fori_loop(1, NCH // RING - 1, body, 0)
        e = NCH - RING
        for j in range(e, NCH):
            slot(j, j % RING, wait_s=True, fire_g=(j + HL < NCH))
        for j in range(NCH - HL, NCH):
            scat_wait(j, j % RING)
        plsc.subcore_barrier()
        pltpu.sync_copy(acc_sh.at[pl.ds(sid * RPT, RPT)],
                        out_hbm.at[cid, pl.ds(sid * RPT, RPT)])


@functools.partial(
    pl.kernel,
    out_type=jax.ShapeDtypeStruct((NW, GPW, L), jnp.float32),
    mesh=_mesh,
    compiler_params=pltpu.CompilerParams(use_tc_tiling_on_sc=False, needs_layout_passes=False),
    scratch_types=[
        pltpu.VMEM((N, 2), jnp.float32),
        pltpu.VMEM((GPW, L), jnp.int32),
        pltpu.VMEM((GPW, L), jnp.int32),
        pltpu.VMEM((GPW, L), jnp.float32),
        pltpu.VMEM((L,), jnp.float32),
    ],
)
def _decode_kernel(ac_hbm, ls_hbm, ld_hbm, bl_hbm, out_hbm,
                   ac_v, ls_v, ld_v, o_v, bl_v):
    cid = lax.axis_index("c")
    sid = lax.axis_index("s")
    wid = sid * NC + cid
    pltpu.sync_copy(ac_hbm, ac_v)
    pltpu.sync_copy(ls_hbm.at[wid], ls_v)
    pltpu.sync_copy(ld_hbm.at[wid], ld_v)
    pltpu.sync_copy(bl_hbm, bl_v)
    bv = bl_v[...]
    col_a = jnp.zeros((L,), jnp.int32)
    col_c = jnp.ones((L,), jnp.int32)

    def body(i, carry):
        j0 = i * 5
        for k in range(5):
            j = j0 + k
            si = ls_v[j]
            di = ld_v[j]
            av = plsc.load_gather(ac_v, [si, col_a])
            cv = plsc.load_gather(ac_v, [di, col_c])
            o_v[j] = av + cv + bv
        return carry

    lax.fori_loop(0, GPW // 5, body, 0)
    pltpu.sync_copy(o_v, out_hbm.at[wid])


def _dinv_of(degp_ref):
    d = degp_ref[...]
    deg = jnp.sum(d[0] + d[1], axis=-1) * (1.0 / DW) + 1.0
    return lax.rsqrt(deg)


def _tca_body(x_ref, pe_ref, w1_ref, degp_ref, ul_ref, ur_ref):
    dinv = _dinv_of(degp_ref)
    xw = jnp.dot(x_ref[...], w1_ref[:D_IN, :], preferred_element_type=jnp.float32)
    xw = xw + jnp.dot(pe_ref[...], w1_ref[D_IN:, :], preferred_element_type=jnp.float32)
    u = xw * dinv[:, None]
    ul_ref[...] = u[:, :DH]
    ur_ref[...] = u[:, DH:]


def _tcb_body(ul_ref, ur_ref, pl_ref, pr_ref, degp_ref, b1_ref, w2_ref,
              ol_ref, or_ref):
    dinv = _dinv_of(degp_ref)
    aggl = ul_ref[...] + pl_ref[0] + pl_ref[1]
    aggr = ur_ref[...] + pr_ref[0] + pr_ref[1]
    agg = jnp.concatenate([aggl, aggr], axis=-1)
    h = jnp.maximum(agg * dinv[:, None] + b1_ref[...][None, :], 0.0)
    u2 = jnp.dot(h, w2_ref[...], preferred_element_type=jnp.float32) * dinv[:, None]
    ol_ref[...] = u2[:, :DH]
    or_ref[...] = u2[:, DH:]


def _tcc_body(ul_ref, ur_ref, pl_ref, pr_ref, degp_ref, b2_ref, wac_ref,
              z_ref, ac_ref):
    dinv = _dinv_of(degp_ref)
    aggl = ul_ref[...] + pl_ref[0] + pl_ref[1]
    aggr = ur_ref[...] + pr_ref[0] + pr_ref[1]
    agg = jnp.concatenate([aggl, aggr], axis=-1)
    z = agg * dinv[:, None] + b2_ref[...][None, :]
    z_ref[...] = z
    ac_ref[...] = jnp.dot(z, wac_ref[...], preferred_element_type=jnp.float32)


BLK = 2000

_tc_a = pl.pallas_call(
    _tca_body,
    grid=(N // BLK,),
    in_specs=[
        pl.BlockSpec((BLK, D_IN), lambda i: (i, 0)),
        pl.BlockSpec((BLK, PDIM), lambda i: (i, 0)),
        pl.BlockSpec((D_IN + PDIM, HID), lambda i: (0, 0)),
        pl.BlockSpec((NC, BLK, DW), lambda i: (0, i, 0)),
    ],
    out_specs=[
        pl.BlockSpec((BLK, DH), lambda i: (i, 0)),
        pl.BlockSpec((BLK, DH), lambda i: (i, 0)),
    ],
    out_shape=[
        jax.ShapeDtypeStruct((N, DH), jnp.float32),
        jax.ShapeDtypeStruct((N, DH), jnp.float32),
    ],
)

_tc_b = pl.pallas_call(
    _tcb_body,
    grid=(N // BLK,),
    in_specs=[
        pl.BlockSpec((BLK, DH), lambda i: (i, 0)),
        pl.BlockSpec((BLK, DH), lambda i: (i, 0)),
        pl.BlockSpec((NC, BLK, DH), lambda i: (0, i, 0)),
        pl.BlockSpec((NC, BLK, DH), lambda i: (0, i, 0)),
        pl.BlockSpec((NC, BLK, DW), lambda i: (0, i, 0)),
        pl.BlockSpec((HID,), lambda i: (0,)),
        pl.BlockSpec((HID, D), lambda i: (0, 0)),
    ],
    out_specs=[
        pl.BlockSpec((BLK, DH), lambda i: (i, 0)),
        pl.BlockSpec((BLK, DH), lambda i: (i, 0)),
    ],
    out_shape=[
        jax.ShapeDtypeStruct((N, DH), jnp.float32),
        jax.ShapeDtypeStruct((N, DH), jnp.float32),
    ],
)

_tc_c = pl.pallas_call(
    _tcc_body,
    grid=(N // BLK,),
    in_specs=[
        pl.BlockSpec((BLK, DH), lambda i: (i, 0)),
        pl.BlockSpec((BLK, DH), lambda i: (i, 0)),
        pl.BlockSpec((NC, BLK, DH), lambda i: (0, i, 0)),
        pl.BlockSpec((NC, BLK, DH), lambda i: (0, i, 0)),
        pl.BlockSpec((NC, BLK, DW), lambda i: (0, i, 0)),
        pl.BlockSpec((D,), lambda i: (0,)),
        pl.BlockSpec((D, 2), lambda i: (0, 0)),
    ],
    out_specs=[
        pl.BlockSpec((BLK, D), lambda i: (i, 0)),
        pl.BlockSpec((BLK, 2), lambda i: (i, 0)),
    ],
    out_shape=[
        jax.ShapeDtypeStruct((N, D), jnp.float32),
        jax.ShapeDtypeStruct((N, 2), jnp.float32),
    ],
)


def kernel(x, edge_index, edge_label_index, pos_emb, W1, b1, W2, b2, Wl, bl):
    src_r = edge_index[0].reshape(NW, NCH, CH)
    dst_r = edge_index[1].reshape(NW, NCH, CH)
    ls_r = edge_label_index[0].reshape(NW, GPW, L)
    ld_r = edge_label_index[1].reshape(NW, GPW, L)
    zeros_half = jnp.zeros((NP, DH), jnp.float32)
    zeros_deg = jnp.zeros((NP, DW), jnp.float32)
    ones_deg = jnp.ones((CH, DW), jnp.float32)
    bl16 = jnp.full((L,), bl[0], jnp.float32)
    wac = jnp.stack([Wl[:HID, 0], Wl[HID:, 0]], axis=1)

    degp = _deg_kernel(dst_r, ones_deg, zeros_deg)
    u1l, u1r = _tc_a(x, pos_emb, W1, degp)
    p1l, p1r = _scatter_kernel(u1l, u1r, src_r, dst_r, zeros_half)
    u2l, u2r = _tc_b(u1l, u1r, p1l, p1r, degp, b1, W2)
    p2l, p2r = _scatter_kernel(u2l, u2r, src_r, dst_r, zeros_half)
    z, ac = _tc_c(u2l, u2r, p2l, p2r, degp, b2, wac)
    ep = _decode_kernel(ac, ls_r, ld_r, bl16).reshape(E)
    return (z, ep)
